# baseline (speedup 1.0000x reference)
import functools

import jax
import jax.numpy as jnp
from jax import lax
from jax.experimental import pallas as pl
from jax.experimental.pallas import tpu as pltpu
from jax.experimental.pallas import tpu_sc as plsc

BATCH = 16384
D = 16
CHUNK = 128


def _build(nc, ns):
    nw = nc * ns
    b_per_w = BATCH // nw
    n_chunks = b_per_w // CHUNK
    mesh = plsc.VectorSubcoreMesh(core_axis_name="c", subcore_axis_name="s")

    @functools.partial(
        pl.kernel,
        mesh=mesh,
        compiler_params=pltpu.CompilerParams(use_tc_tiling_on_sc=False),
        out_type=jax.ShapeDtypeStruct((BATCH, 2, D), jnp.float32),
        scratch_types=[
            pltpu.VMEM((n_chunks, CHUNK), jnp.int32),
            pltpu.VMEM((n_chunks, CHUNK), jnp.int32),
            pltpu.VMEM((b_per_w, D), jnp.float32),
            pltpu.VMEM((b_per_w, D), jnp.float32),
            pltpu.SemaphoreType.DMA,
        ],
    )
    def k(id_hbm, gics_hbm, tid_hbm, tgics_hbm, out_hbm,
          idv, gidv, rid, rgics, sem):
        wid = lax.axis_index("s") * nc + lax.axis_index("c")
        base = wid * b_per_w
        pltpu.sync_copy(id_hbm.at[pl.ds(wid * n_chunks, n_chunks)], idv)
        pltpu.sync_copy(gics_hbm.at[pl.ds(wid * n_chunks, n_chunks)], gidv)
        copies = []
        for j in range(n_chunks):
            copies.append(pltpu.async_copy(
                tid_hbm.at[idv.at[j]],
                rid.at[pl.ds(j * CHUNK, CHUNK)], sem))
            copies.append(pltpu.async_copy(
                tgics_hbm.at[gidv.at[j]],
                rgics.at[pl.ds(j * CHUNK, CHUNK)], sem))
        for c in copies:
            c.wait()
        pltpu.sync_copy(rid, out_hbm.at[pl.ds(base, b_per_w), 0])
        pltpu.sync_copy(rgics, out_hbm.at[pl.ds(base, b_per_w), 1])

    return k


def kernel(item_id, item_gics, table_item_id, table_item_gics):
    info = plsc.get_sparse_core_info()
    k = _build(info.num_cores, info.num_subcores)
    id2d = item_id.reshape(-1, CHUNK)
    gics2d = item_gics.reshape(-1, CHUNK)
    out = k(id2d, gics2d, table_item_id, table_item_gics)
    return out.reshape(BATCH, 2 * D)


# speedup vs baseline: 3.6065x; 3.6065x over previous
import functools

import jax
import jax.numpy as jnp
from jax import lax
from jax.experimental import pallas as pl
from jax.experimental.pallas import tpu as pltpu
from jax.experimental.pallas import tpu_sc as plsc

BATCH = 16384
D = 16
V_ID = 100001
V_GICS = 1001
CHUNK = 8192
LANES = 16


def _build(nc, ns):
    mesh = plsc.VectorSubcoreMesh(core_axis_name="c", subcore_axis_name="s")

    @functools.partial(
        pl.kernel,
        mesh=mesh,
        compiler_params=pltpu.CompilerParams(
            use_tc_tiling_on_sc=True, needs_layout_passes=False),
        out_type=jax.ShapeDtypeStruct((2 * D, BATCH), jnp.float32),
        scratch_types=[
            pltpu.VMEM((V_ID,), jnp.float32),
            pltpu.VMEM((V_GICS,), jnp.float32),
            pltpu.VMEM((CHUNK,), jnp.int32),
            pltpu.VMEM((CHUNK,), jnp.float32),
        ],
    )
    def k(id_hbm, gics_hbm, tid_hbm, tgics_hbm, out_hbm,
          src_id, src_gics, idx_v, out_v):
        wid = lax.axis_index("s") * nc + lax.axis_index("c")

        def gather_row(src_ref, idx_hbm):
            for c in range(BATCH // CHUNK):
                pltpu.sync_copy(idx_hbm.at[pl.ds(c * CHUNK, CHUNK)], idx_v)

                def body(g, _):
                    iv = idx_v[pl.ds(g * LANES, LANES)]
                    out_v[pl.ds(g * LANES, LANES)] = plsc.load_gather(
                        src_ref, [iv])
                    return 0

                lax.fori_loop(0, CHUNK // LANES, body, 0, unroll=8)
                pltpu.sync_copy(out_v, out_hbm.at[wid, pl.ds(c * CHUNK, CHUNK)])

        @pl.when(wid < D)
        def _():
            pltpu.sync_copy(tid_hbm.at[wid], src_id)
            gather_row(src_id, id_hbm)

        @pl.when(wid >= D)
        def _():
            pltpu.sync_copy(tgics_hbm.at[wid - D], src_gics)
            gather_row(src_gics, gics_hbm)

    return k


def kernel(item_id, item_gics, table_item_id, table_item_gics):
    info = plsc.get_sparse_core_info()
    k = _build(info.num_cores, info.num_subcores)
    out_t = k(item_id, item_gics, table_item_id.T, table_item_gics.T)
    return out_t.T


# speedup vs baseline: 3.6801x; 1.0204x over previous
import functools

import jax
import jax.numpy as jnp
from jax import lax
from jax.experimental import pallas as pl
from jax.experimental.pallas import tpu as pltpu
from jax.experimental.pallas import tpu_sc as plsc

BATCH = 16384
D = 16
V_ID = 100001
V_GICS = 1001
CHUNK = 4096
NCHUNK = BATCH // CHUNK
LANES = 16


def _build(nc, ns):
    mesh = plsc.VectorSubcoreMesh(core_axis_name="c", subcore_axis_name="s")

    @functools.partial(
        pl.kernel,
        mesh=mesh,
        compiler_params=pltpu.CompilerParams(
            use_tc_tiling_on_sc=True, needs_layout_passes=False),
        out_type=jax.ShapeDtypeStruct((2 * D, BATCH), jnp.float32),
        scratch_types=[
            pltpu.VMEM((V_ID,), jnp.float32),
            pltpu.VMEM((V_GICS,), jnp.float32),
            pltpu.VMEM((2, CHUNK), jnp.int32),
            pltpu.VMEM((2, CHUNK), jnp.float32),
            pltpu.SemaphoreType.DMA,
            pltpu.SemaphoreType.DMA,
            pltpu.SemaphoreType.DMA,
            pltpu.SemaphoreType.DMA,
            pltpu.SemaphoreType.DMA,
        ],
    )
    def k(id_hbm, gics_hbm, tid_hbm, tgics_hbm, out_hbm,
          src_id, src_gics, idx_v, out_v, s_tab, s_i0, s_i1, s_o0, s_o1):
        wid = lax.axis_index("s") * nc + lax.axis_index("c")
        s_idx = (s_i0, s_i1)
        s_out = (s_o0, s_o1)

        def gather_row(src_ref, idx_hbm, stage):
            idx_cp = [None] * NCHUNK
            idx_cp[0] = pltpu.async_copy(
                idx_hbm.at[pl.ds(0, CHUNK)], idx_v.at[0], s_idx[0])
            stage.wait()
            out_cp = [None, None]
            for c in range(NCHUNK):
                buf = c % 2
                if c + 1 < NCHUNK:
                    idx_cp[c + 1] = pltpu.async_copy(
                        idx_hbm.at[pl.ds((c + 1) * CHUNK, CHUNK)],
                        idx_v.at[(c + 1) % 2], s_idx[(c + 1) % 2])
                idx_cp[c].wait()
                if out_cp[buf] is not None:
                    out_cp[buf].wait()

                def body(g, _):
                    iv = idx_v[buf, pl.ds(g * LANES, LANES)]
                    out_v[buf, pl.ds(g * LANES, LANES)] = plsc.load_gather(
                        src_ref, [iv])
                    return 0

                lax.fori_loop(0, CHUNK // LANES, body, 0, unroll=8)
                out_cp[buf] = pltpu.async_copy(
                    out_v.at[buf], out_hbm.at[wid, pl.ds(c * CHUNK, CHUNK)],
                    s_out[buf])
            for buf in range(2):
                if out_cp[buf] is not None:
                    out_cp[buf].wait()

        @pl.when(wid < D)
        def _():
            stage = pltpu.async_copy(tid_hbm.at[wid], src_id, s_tab)
            gather_row(src_id, id_hbm, stage)

        @pl.when(wid >= D)
        def _():
            stage = pltpu.async_copy(tgics_hbm.at[wid - D], src_gics, s_tab)
            gather_row(src_gics, gics_hbm, stage)

    return k


def kernel(item_id, item_gics, table_item_id, table_item_gics):
    info = plsc.get_sparse_core_info()
    k = _build(info.num_cores, info.num_subcores)
    out_t = k(item_id, item_gics, table_item_id.T, table_item_gics.T)
    return out_t.T


# speedup vs baseline: 4.5178x; 1.2276x over previous
import functools

import jax
import jax.numpy as jnp
from jax import lax
from jax.experimental import pallas as pl
from jax.experimental.pallas import tpu as pltpu
from jax.experimental.pallas import tpu_sc as plsc

BATCH = 16384
D = 16
V_ID = 100001
V_GICS = 1001
CHUNK = 4096
NCHUNK = BATCH // CHUNK
LANES = 16


def _build(nc, ns):
    mesh = plsc.VectorSubcoreMesh(core_axis_name="c", subcore_axis_name="s")

    @functools.partial(
        pl.kernel,
        mesh=mesh,
        compiler_params=pltpu.CompilerParams(
            use_tc_tiling_on_sc=True, needs_layout_passes=False),
        out_type=jax.ShapeDtypeStruct((2 * D, BATCH), jnp.float32),
        scratch_types=[
            pltpu.VMEM((V_ID,), jnp.float32),
            pltpu.VMEM((V_GICS,), jnp.float32),
            pltpu.VMEM((2, CHUNK), jnp.int32),
            pltpu.VMEM((2, CHUNK), jnp.float32),
            pltpu.SemaphoreType.DMA,
            pltpu.SemaphoreType.DMA,
            pltpu.SemaphoreType.DMA,
            pltpu.SemaphoreType.DMA,
            pltpu.SemaphoreType.DMA,
        ],
    )
    def k(id_hbm, gics_hbm, tid_hbm, tgics_hbm, out_hbm,
          src_id, src_gics, idx_v, out_v, s_tab, s_i0, s_i1, s_o0, s_o1):
        wid = lax.axis_index("s") * nc + lax.axis_index("c")
        s_idx = (s_i0, s_i1)
        s_out = (s_o0, s_o1)

        def gather_row(src_ref, idx_hbm, stage):
            idx_cp = [None] * NCHUNK
            idx_cp[0] = pltpu.async_copy(
                idx_hbm.at[pl.ds(0, CHUNK)], idx_v.at[0], s_idx[0])
            stage.wait()
            out_cp = [None, None]
            for c in range(NCHUNK):
                buf = c % 2
                if c + 1 < NCHUNK:
                    idx_cp[c + 1] = pltpu.async_copy(
                        idx_hbm.at[pl.ds((c + 1) * CHUNK, CHUNK)],
                        idx_v.at[(c + 1) % 2], s_idx[(c + 1) % 2])
                idx_cp[c].wait()
                if out_cp[buf] is not None:
                    out_cp[buf].wait()

                @plsc.parallel_loop(0, CHUNK, LANES, unroll=8)
                def body(g):
                    iv = idx_v[buf, pl.ds(g, LANES)]
                    out_v[buf, pl.ds(g, LANES)] = plsc.load_gather(
                        src_ref, [iv])
                out_cp[buf] = pltpu.async_copy(
                    out_v.at[buf], out_hbm.at[wid, pl.ds(c * CHUNK, CHUNK)],
                    s_out[buf])
            for buf in range(2):
                if out_cp[buf] is not None:
                    out_cp[buf].wait()

        @pl.when(wid < D)
        def _():
            stage = pltpu.async_copy(tid_hbm.at[wid], src_id, s_tab)
            gather_row(src_id, id_hbm, stage)

        @pl.when(wid >= D)
        def _():
            stage = pltpu.async_copy(tgics_hbm.at[wid - D], src_gics, s_tab)
            gather_row(src_gics, gics_hbm, stage)

    return k


def kernel(item_id, item_gics, table_item_id, table_item_gics):
    info = plsc.get_sparse_core_info()
    k = _build(info.num_cores, info.num_subcores)
    out_t = k(item_id, item_gics, table_item_id.T, table_item_gics.T)
    return out_t.T
